# baseline (speedup 1.0000x reference)
import functools

import jax
import jax.numpy as jnp
from jax import lax
from jax.experimental import pallas as pl
from jax.experimental.pallas import tpu as pltpu
from jax.experimental.pallas import tpu_sc as plsc

_NBUF = 2
_CS = 8


@functools.lru_cache(maxsize=None)
def _make_gather(V, D, NS_TOT, T):
  info = plsc.get_sparse_core_info()
  NC, NS = info.num_cores, info.num_subcores
  NW = NC * NS
  assert NS_TOT % NW == 0
  s_per_w = NS_TOT // NW
  b_per_w = s_per_w * T
  CS = _CS
  C = CS * T
  NBUF = _NBUF
  assert s_per_w % CS == 0
  n_chunks = s_per_w // CS
  assert b_per_w + NBUF * C * D <= 131000

  mesh = plsc.VectorSubcoreMesh(core_axis_name="c", subcore_axis_name="s")

  @functools.partial(
      pl.kernel,
      mesh=mesh,
      out_type=jax.ShapeDtypeStruct((NS_TOT, T, D), jnp.float32),
      scratch_types=[
          pltpu.VMEM((b_per_w,), jnp.int32),
          pltpu.VMEM((NBUF, C, D), jnp.float32),
          pltpu.SemaphoreType.DMA((NBUF,)),
          pltpu.SemaphoreType.DMA((NBUF,)),
      ],
      compiler_params=pltpu.CompilerParams(use_tc_tiling_on_sc=False),
  )
  def k(x_hbm, w_hbm, out_hbm, idx_v, rows_v, gsem, ssem):
    wid = lax.axis_index("s") * NC + lax.axis_index("c")
    sent0 = wid * s_per_w
    pltpu.sync_copy(x_hbm.at[pl.ds(sent0 * T, b_per_w)], idx_v)

    def gather(c, b):
      return pltpu.async_copy(
          w_hbm.at[idx_v.at[pl.ds(c * C, C)]], rows_v.at[b], gsem.at[b])

    def store(c, b):
      return [
          pltpu.async_copy(
              rows_v.at[b].at[pl.ds(t * T, T)],
              out_hbm.at[sent0 + c * CS + t],
              ssem.at[b],
          )
          for t in range(CS)
      ]

    g = [None] * NBUF
    s = [None] * NBUF
    for c in range(min(NBUF, n_chunks)):
      g[c] = gather(c, c)
    for c in range(n_chunks):
      b = c % NBUF
      g[b].wait()
      s[b] = store(c, b)
      nxt = c + NBUF
      if nxt < n_chunks:
        for cp in s[b]:
          cp.wait()
        g[b] = gather(nxt, b)
    for c in range(max(0, n_chunks - NBUF), n_chunks):
      for cp in s[c % NBUF]:
        cp.wait()

  return k


def kernel(x, weight):
  NS_TOT, T = x.shape
  V, D = weight.shape
  xf = x.reshape(NS_TOT * T).astype(jnp.int32)
  return _make_gather(V, D, NS_TOT, T)(xf, weight)


# speedup vs baseline: 1.0026x; 1.0026x over previous
import functools

import jax
import jax.numpy as jnp
from jax import lax
from jax.experimental import pallas as pl
from jax.experimental.pallas import tpu as pltpu
from jax.experimental.pallas import tpu_sc as plsc

_NBUF = 3
_CHUNK = 1024


@functools.lru_cache(maxsize=None)
def _make_gather(V, D, B):
  info = plsc.get_sparse_core_info()
  NC, NS = info.num_cores, info.num_subcores
  NW = NC * NS
  assert B % NW == 0
  b_per_w = B // NW
  C = _CHUNK
  NBUF = _NBUF
  assert b_per_w % C == 0
  n_chunks = b_per_w // C
  assert b_per_w + NBUF * C * D <= 131000

  mesh = plsc.VectorSubcoreMesh(core_axis_name="c", subcore_axis_name="s")

  @functools.partial(
      pl.kernel,
      mesh=mesh,
      out_type=jax.ShapeDtypeStruct((B, D), jnp.float32),
      scratch_types=[
          pltpu.VMEM((b_per_w,), jnp.int32),
          pltpu.VMEM((NBUF, C, D), jnp.float32),
          pltpu.SemaphoreType.DMA((NBUF,)),
          pltpu.SemaphoreType.DMA((NBUF,)),
      ],
      compiler_params=pltpu.CompilerParams(use_tc_tiling_on_sc=False),
  )
  def k(x_hbm, w_hbm, out_hbm, idx_v, rows_v, gsem, ssem):
    w2d = w_hbm
    out2d = out_hbm
    wid = lax.axis_index("s") * NC + lax.axis_index("c")
    base0 = wid * b_per_w
    pltpu.sync_copy(x_hbm.at[pl.ds(base0, b_per_w)], idx_v)

    def gather(c, b):
      return pltpu.async_copy(
          w2d.at[idx_v.at[pl.ds(c * C, C)]], rows_v.at[b], gsem.at[b])

    def store(c, b):
      return pltpu.async_copy(
          rows_v.at[b], out2d.at[pl.ds(base0 + c * C, C)], ssem.at[b])

    g = [None] * NBUF
    s = [None] * NBUF
    for c in range(min(NBUF, n_chunks)):
      g[c] = gather(c, c)
    for c in range(n_chunks):
      b = c % NBUF
      g[b].wait()
      s[b] = store(c, b)
      nxt = c + NBUF
      if nxt < n_chunks:
        s[b].wait()
        g[b] = gather(nxt, b)
    for c in range(max(0, n_chunks - NBUF), n_chunks):
      s[c % NBUF].wait()

  return k


def kernel(x, weight):
  NS_TOT, T = x.shape
  V, D = weight.shape
  B = NS_TOT * T
  xf = x.reshape(B).astype(jnp.int32)
  outlin = _make_gather(V, D, B)(xf, weight)
  return outlin.reshape(NS_TOT, T, D)


# speedup vs baseline: 1.0031x; 1.0005x over previous
import functools

import jax
import jax.numpy as jnp
from jax import lax
from jax.experimental import pallas as pl
from jax.experimental.pallas import tpu as pltpu
from jax.experimental.pallas import tpu_sc as plsc

_NBUF = 2
_CHUNK = 1600


@functools.lru_cache(maxsize=None)
def _make_gather(V, D, B):
  info = plsc.get_sparse_core_info()
  NC, NS = info.num_cores, info.num_subcores
  NW = NC * NS
  assert B % NW == 0
  b_per_w = B // NW
  C = _CHUNK
  NBUF = _NBUF
  assert b_per_w % C == 0
  n_chunks = b_per_w // C
  assert b_per_w + NBUF * C * D <= 131000

  mesh = plsc.VectorSubcoreMesh(core_axis_name="c", subcore_axis_name="s")

  @functools.partial(
      pl.kernel,
      mesh=mesh,
      out_type=jax.ShapeDtypeStruct((B, D), jnp.float32),
      scratch_types=[
          pltpu.VMEM((b_per_w,), jnp.int32),
          pltpu.VMEM((NBUF, C, D), jnp.float32),
          pltpu.SemaphoreType.DMA((NBUF,)),
          pltpu.SemaphoreType.DMA((NBUF,)),
      ],
      compiler_params=pltpu.CompilerParams(use_tc_tiling_on_sc=False),
  )
  def k(x_hbm, w_hbm, out_hbm, idx_v, rows_v, gsem, ssem):
    w2d = w_hbm
    out2d = out_hbm
    wid = lax.axis_index("s") * NC + lax.axis_index("c")
    base0 = wid * b_per_w
    pltpu.sync_copy(x_hbm.at[pl.ds(base0, b_per_w)], idx_v)

    def gather(c, b):
      return pltpu.async_copy(
          w2d.at[idx_v.at[pl.ds(c * C, C)]], rows_v.at[b], gsem.at[b])

    def store(c, b):
      return pltpu.async_copy(
          rows_v.at[b], out2d.at[pl.ds(base0 + c * C, C)], ssem.at[b])

    g = [None] * NBUF
    s = [None] * NBUF
    for c in range(min(NBUF, n_chunks)):
      g[c] = gather(c, c)
    for c in range(n_chunks):
      b = c % NBUF
      g[b].wait()
      s[b] = store(c, b)
      nxt = c + NBUF
      if nxt < n_chunks:
        s[b].wait()
        g[b] = gather(nxt, b)
    for c in range(max(0, n_chunks - NBUF), n_chunks):
      s[c % NBUF].wait()

  return k


def kernel(x, weight):
  NS_TOT, T = x.shape
  V, D = weight.shape
  B = NS_TOT * T
  xf = x.reshape(B).astype(jnp.int32)
  outlin = _make_gather(V, D, B)(xf, weight)
  return outlin.reshape(NS_TOT, T, D)
